# Initial kernel scaffold; baseline (speedup 1.0000x reference)
import jax
import jax.numpy as jnp
from jax.experimental import pallas as pl


def kernel(input, edge, edge_embed, a, a_2):
    raise NotImplementedError("write your pallas kernel here")


# speedup vs baseline: 1.9831x; 1.9831x over previous
import functools

import jax
import jax.numpy as jnp
from jax import lax
from jax.experimental import pallas as pl
from jax.experimental.pallas import tpu as pltpu
from jax.experimental.pallas import tpu_sc as plsc

N = 10000
E = 320000
DIN = 128
DOUT = 128
ALPHA = 0.2
NPAD = 10240
W = 144
NCORES = 2
NSUB = 16
NTILES = NCORES * NSUB
EPT = E // NTILES
B = 80
GROUPS = B // 16
CHUNKS = EPT // B
ROWS_PER_TILE = NPAD // NSUB


def _pre_body(x_ref, a_ref, a2r_ref, h1_ref, s_ref, v2_ref):
    x = x_ref[...]
    a1 = a_ref[:, :DIN]
    a2 = a_ref[:, DIN:]
    a2r = a2r_ref[...]
    h1 = lax.dot_general(x, a1, (((1,), (1,)), ((), ())),
                         preferred_element_type=jnp.float32)
    h1_ref[...] = h1
    s_ref[...] = lax.dot_general(h1, a2r, (((1,), (1,)), ((), ())),
                                 preferred_element_type=jnp.float32)
    v2_ref[...] = lax.dot_general(a2r, a2, (((1,), (0,)), ((), ())),
                                  preferred_element_type=jnp.float32)


def _edge_body(emb_hbm, src_hbm, s_hbm, v2_hbm, q_hbm,
               embbuf, wbuf, srcbuf, s_local, v2buf, q_shared):
    cid = lax.axis_index("c")
    sid = lax.axis_index("s")
    wid = cid * NSUB + sid

    def _zrow(i, carry):
        for j in range(W // 16):
            wbuf[i, pl.ds(j * 16, 16)] = jnp.zeros((16,), jnp.float32)
        return carry
    lax.fori_loop(0, B, _zrow, 0)
    for k in range(ROWS_PER_TILE // B):
        pltpu.sync_copy(wbuf, q_shared.at[pl.ds(sid * ROWS_PER_TILE + k * B, B)])

    pltpu.sync_copy(s_hbm, s_local)
    pltpu.sync_copy(v2_hbm, v2buf)
    plsc.subcore_barrier()

    ebase = wid * EPT
    iota16 = lax.iota(jnp.int32, 16)
    lane0 = iota16 == 0

    def _chunk(c, carry):
        off = ebase + c * B
        pltpu.sync_copy(emb_hbm.at[pl.ds(off * DOUT, B * DOUT)], embbuf)
        pltpu.sync_copy(src_hbm.at[pl.ds(off, B)], srcbuf.at[0])

        def _group(g, gcarry):
            gb = g * 16
            src_vec = srcbuf[0, pl.ds(gb, 16)]
            s_vec = plsc.load_gather(s_local, [src_vec])
            flatbase = (gb + iota16) * DOUT
            acc = jnp.zeros((16,), jnp.float32)
            for jj in range(DOUT // 16):
                v2c = v2buf[pl.ds(jj * 16, 16)]
                for m in range(16):
                    v = plsc.load_gather(embbuf, [flatbase + (jj * 16 + m)])
                    acc = acc + v * v2c[m]
            p = s_vec + acc
            nl = jnp.where(p >= 0.0, p, p * ALPHA)
            e_vec = jnp.exp(-nl)
            for l in range(16):
                e_l = e_vec[l]
                rowb = (gb + l) * DOUT
                for j in range(DOUT // 16):
                    wbuf[gb + l, pl.ds(j * 16, 16)] = (
                        embbuf[pl.ds(rowb + j * 16, 16)] * e_l)
                wbuf[gb + l, pl.ds(DOUT, 16)] = jnp.where(lane0, e_l, 0.0)
            return gcarry
        lax.fori_loop(0, GROUPS, _group, 0)

        pltpu.sync_copy(wbuf, q_shared.at[srcbuf.at[0]], add=True)
        return carry
    lax.fori_loop(0, CHUNKS, _chunk, 0)

    plsc.subcore_barrier()
    pltpu.sync_copy(q_shared.at[pl.ds(sid * ROWS_PER_TILE, ROWS_PER_TILE)],
                    q_hbm.at[cid, pl.ds(sid * ROWS_PER_TILE, ROWS_PER_TILE)])


def _post_body(q_ref, h1_ref, a_ref, o_ref):
    a2 = a_ref[:, DIN:]
    qs = q_ref[0] + q_ref[1]
    qm = qs[:, :DOUT]
    r = qs[:, DOUT:DOUT + 1]
    num = r * h1_ref[...] + lax.dot_general(
        qm, a2, (((1,), (1,)), ((), ())), preferred_element_type=jnp.float32)
    rt = jnp.where(r == 0.0, 1e-12, r)
    o_ref[...] = num / rt


def kernel(input, edge, edge_embed, a, a_2):
    src = edge[0, :]
    xp = jnp.pad(input, ((0, NPAD - N), (0, 0)))

    h1p, s_col, v2row = pl.pallas_call(
        _pre_body,
        out_shape=[
            jax.ShapeDtypeStruct((NPAD, DOUT), jnp.float32),
            jax.ShapeDtypeStruct((NPAD, 1), jnp.float32),
            jax.ShapeDtypeStruct((1, DOUT), jnp.float32),
        ],
    )(xp, a, a_2)

    edge_pass = functools.partial(
        pl.kernel,
        out_type=jax.ShapeDtypeStruct((NCORES, NPAD, W), jnp.float32),
        mesh=plsc.VectorSubcoreMesh(
            core_axis_name="c", subcore_axis_name="s",
            num_cores=NCORES, num_subcores=NSUB),
        compiler_params=pltpu.CompilerParams(
            needs_layout_passes=False, use_tc_tiling_on_sc=False),
        scratch_types=[
            pltpu.VMEM((B * DOUT,), jnp.float32),
            pltpu.VMEM((B, W), jnp.float32),
            pltpu.VMEM((1, B), jnp.int32),
            pltpu.VMEM((NPAD,), jnp.float32),
            pltpu.VMEM((DOUT,), jnp.float32),
            pltpu.VMEM_SHARED((NPAD, W), jnp.float32),
        ],
    )(_edge_body)
    q_all = edge_pass(edge_embed.reshape(E * DOUT), src,
                      s_col.reshape(NPAD), v2row.reshape(DOUT))

    out = pl.pallas_call(
        _post_body,
        out_shape=jax.ShapeDtypeStruct((NPAD, DOUT), jnp.float32),
    )(q_all, h1p, a)
    return out[:N]


# speedup vs baseline: 6.0050x; 3.0281x over previous
import functools

import jax
import jax.numpy as jnp
from jax import lax
from jax.experimental import pallas as pl
from jax.experimental.pallas import tpu as pltpu
from jax.experimental.pallas import tpu_sc as plsc

N = 10000
E = 320000
DIN = 128
DOUT = 128
ALPHA = 0.2
NPAD = 10240
NCORES = 2
NSUB = 16
NTILES = NCORES * NSUB
EPT = E // NTILES
B = 80
GROUPS = B // 16
CHUNKS = EPT // B
NBUF = 3
MAIN = (CHUNKS // NBUF) * NBUF
ROWS_PER_TILE = NPAD // NSUB


def _pre_body(x_ref, a_ref, a2r_ref, h1_ref, s_ref):
    x = x_ref[...]
    a1 = a_ref[:, :DIN]
    a2r = a2r_ref[...]
    h1 = lax.dot_general(x, a1, (((1,), (1,)), ((), ())),
                         preferred_element_type=jnp.float32)
    h1_ref[...] = h1
    s_ref[...] = lax.dot_general(h1, a2r, (((1,), (1,)), ((), ())),
                                 preferred_element_type=jnp.float32)


def _tmatvec_body(emb_ref, a_ref, a2r_ref, t_ref):
    a2 = a_ref[:, DIN:]
    v2 = lax.dot_general(a2r_ref[...], a2, (((1,), (0,)), ((), ())),
                         preferred_element_type=jnp.float32)
    t_ref[...] = lax.dot_general(emb_ref[...], v2, (((1,), (1,)), ((), ())),
                                 preferred_element_type=jnp.float32)


def _edge_body(emb_hbm, src_hbm, s_hbm, t_hbm, zq_hbm, zr_hbm,
               q_hbm, r_hbm,
               embbuf, srcbuf, tchunk, ebuf, s_local,
               q_shared, r_shared,
               isem0, isem1, isem2, ssem0, ssem1, ssem2):
    isems = (isem0, isem1, isem2)
    ssems = (ssem0, ssem1, ssem2)
    cid = lax.axis_index("c")
    sid = lax.axis_index("s")
    wid = cid * NSUB + sid
    ebase = wid * EPT
    iota16 = lax.iota(jnp.int32, 16)

    def _in_copies(c, p):
        off = ebase + c * B
        return (
            pltpu.make_async_copy(emb_hbm.at[pl.ds(off, B)], embbuf.at[p],
                                  isems[p]),
            pltpu.make_async_copy(src_hbm.at[pl.ds(off, B)], srcbuf.at[p, 0],
                                  isems[p]),
            pltpu.make_async_copy(t_hbm.at[pl.ds(off, B)], tchunk.at[p],
                                  isems[p]),
        )

    def _issue_in(c, p):
        for d in _in_copies(c, p):
            d.start()

    def _wait_in(c, p):
        for d in _in_copies(c, p):
            d.wait()

    def _scat_copies(p):
        return (
            pltpu.make_async_copy(embbuf.at[p], q_shared.at[srcbuf.at[p, 0]],
                                  ssems[p]),
            pltpu.make_async_copy(ebuf.at[p], r_shared.at[srcbuf.at[p, 0]],
                                  ssems[p]),
        )

    def _issue_scat(p):
        for d in _scat_copies(p):
            d.start(add=True)

    def _wait_scat(p):
        for d in _scat_copies(p):
            d.wait()

    _issue_in(0, 0)
    _issue_in(1, 1)
    pltpu.sync_copy(zq_hbm, q_shared.at[pl.ds(sid * ROWS_PER_TILE,
                                              ROWS_PER_TILE)])
    pltpu.sync_copy(zr_hbm, r_shared.at[pl.ds(sid * ROWS_PER_TILE,
                                              ROWS_PER_TILE)])
    pltpu.sync_copy(s_hbm, s_local)
    plsc.subcore_barrier()

    def _compute(c, p):
        @plsc.parallel_loop(0, GROUPS, unroll=GROUPS)
        def _group(g):
            gb = g * 16
            src_vec = srcbuf[p, 0, pl.ds(gb, 16)]
            s_vec = plsc.load_gather(s_local, [src_vec])
            t_vec = tchunk[p, pl.ds(gb, 16)]
            pv = s_vec + t_vec
            nl = jnp.where(pv >= 0.0, pv, pv * ALPHA)
            e_vec = jnp.exp(-nl)
            ebuf[p, pl.ds(gb, 16)] = e_vec
            for l in range(16):
                e_l = e_vec[l]
                for j in range(DOUT // 16):
                    embbuf[p, gb + l, pl.ds(j * 16, 16)] = (
                        embbuf[p, gb + l, pl.ds(j * 16, 16)] * e_l)

    def _chunk(c, p):
        _wait_in(c, p)
        _compute(c, p)
        _issue_scat(p)
        pprev = (p + 2) % NBUF
        @pl.when(c >= 1)
        def _():
            _wait_scat(pprev)
        _issue_in(c + 2, pprev)

    def _main(i, carry):
        for p in range(NBUF):
            _chunk(i * NBUF + p, p)
        return carry
    lax.fori_loop(0, MAIN // NBUF, _main, 0)

    for c in range(MAIN, CHUNKS):
        p = c % NBUF
        _wait_in(c, p)
        _compute(c, p)
        _wait_scat((p + 2) % NBUF)
        _issue_scat(p)
    _wait_scat((CHUNKS - 1) % NBUF)

    plsc.subcore_barrier()
    rows = pl.ds(sid * ROWS_PER_TILE, ROWS_PER_TILE)
    pltpu.sync_copy(q_shared.at[rows], q_hbm.at[cid, rows])
    pltpu.sync_copy(r_shared.at[rows], r_hbm.at[cid, rows])


def _post_body(q_ref, r_ref, h1_ref, a_ref, o_ref):
    a2 = a_ref[:, DIN:]
    qs = q_ref[0] + q_ref[1]
    r = r_ref[0] + r_ref[1]
    num = r * h1_ref[...] + lax.dot_general(
        qs, a2, (((1,), (1,)), ((), ())), preferred_element_type=jnp.float32)
    rt = jnp.where(r == 0.0, 1e-12, r)
    o_ref[...] = num / rt


def kernel(input, edge, edge_embed, a, a_2):
    src = edge[0, :]
    xp = jnp.pad(input, ((0, NPAD - N), (0, 0)))

    h1p, s_col = pl.pallas_call(
        _pre_body,
        out_shape=[
            jax.ShapeDtypeStruct((NPAD, DOUT), jnp.float32),
            jax.ShapeDtypeStruct((NPAD, 1), jnp.float32),
        ],
    )(xp, a, a_2)

    EB = 6400
    t_col = pl.pallas_call(
        _tmatvec_body,
        grid=(E // EB,),
        in_specs=[
            pl.BlockSpec((EB, DOUT), lambda i: (i, 0)),
            pl.BlockSpec((DIN, DIN + DOUT), lambda i: (0, 0)),
            pl.BlockSpec((1, DOUT), lambda i: (0, 0)),
        ],
        out_specs=pl.BlockSpec((EB, 1), lambda i: (i, 0)),
        out_shape=jax.ShapeDtypeStruct((E, 1), jnp.float32),
    )(edge_embed, a, a_2)

    zq = jnp.zeros((ROWS_PER_TILE, DOUT), jnp.float32)
    zr = jnp.zeros((ROWS_PER_TILE,), jnp.float32)

    edge_pass = functools.partial(
        pl.kernel,
        out_type=[
            jax.ShapeDtypeStruct((NCORES, NPAD, DOUT), jnp.float32),
            jax.ShapeDtypeStruct((NCORES, NPAD), jnp.float32),
        ],
        mesh=plsc.VectorSubcoreMesh(
            core_axis_name="c", subcore_axis_name="s",
            num_cores=NCORES, num_subcores=NSUB),
        compiler_params=pltpu.CompilerParams(
            needs_layout_passes=False, use_tc_tiling_on_sc=False),
        scratch_types=[
            pltpu.VMEM((NBUF, B, DOUT), jnp.float32),
            pltpu.VMEM((NBUF, 1, B), jnp.int32),
            pltpu.VMEM((NBUF, B), jnp.float32),
            pltpu.VMEM((NBUF, B), jnp.float32),
            pltpu.VMEM((NPAD,), jnp.float32),
            pltpu.VMEM_SHARED((NPAD, DOUT), jnp.float32),
            pltpu.VMEM_SHARED((NPAD,), jnp.float32),
            pltpu.SemaphoreType.DMA,
            pltpu.SemaphoreType.DMA,
            pltpu.SemaphoreType.DMA,
            pltpu.SemaphoreType.DMA,
            pltpu.SemaphoreType.DMA,
            pltpu.SemaphoreType.DMA,
        ],
    )(_edge_body)
    q_all, r_all = edge_pass(edge_embed, src, s_col.reshape(NPAD),
                             t_col.reshape(E), zq, zr)

    out = pl.pallas_call(
        _post_body,
        out_shape=jax.ShapeDtypeStruct((NPAD, DOUT), jnp.float32),
    )(q_all, r_all.reshape(NCORES, NPAD, 1), h1p, a)
    return out[:N]


# speedup vs baseline: 7.4370x; 1.2385x over previous
import functools

import jax
import jax.numpy as jnp
from jax import lax
from jax.experimental import pallas as pl
from jax.experimental.pallas import tpu as pltpu
from jax.experimental.pallas import tpu_sc as plsc

N = 10000
E = 320000
DIN = 128
DOUT = 128
ALPHA = 0.2
NPAD = 10240
NCORES = 2
NSUB = 16
NTILES = NCORES * NSUB
EPT = E // NTILES
B = 80
GROUPS = B // 16
CHUNKS = EPT // B
NBUF = 3
MAIN = (CHUNKS // NBUF) * NBUF
ROWS_PER_TILE = NPAD // NSUB


def _pre_body(x_ref, a_ref, a2r_ref, h1_ref, s_ref):
    x = x_ref[...]
    a1 = a_ref[:, :DIN]
    a2r = a2r_ref[...]
    h1 = lax.dot_general(x, a1, (((1,), (1,)), ((), ())),
                         preferred_element_type=jnp.float32)
    h1_ref[...] = h1
    s_ref[...] = lax.dot_general(h1, a2r, (((1,), (1,)), ((), ())),
                                 preferred_element_type=jnp.float32)


def _tmatvec_body(emb_ref, a_ref, a2r_ref, t_ref):
    a2 = a_ref[:, DIN:]
    v2 = lax.dot_general(a2r_ref[...], a2, (((1,), (0,)), ((), ())),
                         preferred_element_type=jnp.float32)
    t_ref[...] = lax.dot_general(v2, emb_ref[...], (((1,), (1,)), ((), ())),
                                 preferred_element_type=jnp.float32)


def _edge_body(emb_hbm, src_hbm, s_hbm, t_hbm, zq_hbm, zr_hbm,
               q_hbm, r_hbm,
               embbuf, srcbuf, tchunk, ebuf, s_local,
               q_shared, r_shared,
               isem0, isem1, isem2, ssem0, ssem1, ssem2):
    isems = (isem0, isem1, isem2)
    ssems = (ssem0, ssem1, ssem2)
    cid = lax.axis_index("c")
    sid = lax.axis_index("s")
    wid = cid * NSUB + sid
    ebase = wid * EPT
    iota16 = lax.iota(jnp.int32, 16)

    def _in_copies(c, p):
        off = ebase + c * B
        return (
            pltpu.make_async_copy(emb_hbm.at[pl.ds(off, B)], embbuf.at[p],
                                  isems[p]),
            pltpu.make_async_copy(src_hbm.at[pl.ds(off, B)], srcbuf.at[p, 0],
                                  isems[p]),
            pltpu.make_async_copy(t_hbm.at[0, pl.ds(off, B)], tchunk.at[p],
                                  isems[p]),
        )

    def _issue_in(c, p):
        for d in _in_copies(c, p):
            d.start()

    def _wait_in(c, p):
        for d in _in_copies(c, p):
            d.wait()

    def _scat_copies(p):
        return (
            pltpu.make_async_copy(embbuf.at[p], q_shared.at[srcbuf.at[p, 0]],
                                  ssems[p]),
            pltpu.make_async_copy(ebuf.at[p], r_shared.at[srcbuf.at[p, 0]],
                                  ssems[p]),
        )

    def _issue_scat(p):
        for d in _scat_copies(p):
            d.start(add=True)

    def _wait_scat(p):
        for d in _scat_copies(p):
            d.wait()

    _issue_in(0, 0)
    _issue_in(1, 1)
    pltpu.sync_copy(zq_hbm, q_shared.at[pl.ds(sid * ROWS_PER_TILE,
                                              ROWS_PER_TILE)])
    pltpu.sync_copy(zr_hbm, r_shared.at[pl.ds(sid * ROWS_PER_TILE,
                                              ROWS_PER_TILE)])
    pltpu.sync_copy(s_hbm, s_local)
    plsc.subcore_barrier()

    def _compute(c, p):
        @plsc.parallel_loop(0, GROUPS, unroll=GROUPS)
        def _group(g):
            gb = g * 16
            src_vec = srcbuf[p, 0, pl.ds(gb, 16)]
            s_vec = plsc.load_gather(s_local, [src_vec])
            t_vec = tchunk[p, pl.ds(gb, 16)]
            pv = s_vec + t_vec
            nl = jnp.where(pv >= 0.0, pv, pv * ALPHA)
            e_vec = jnp.exp(-nl)
            ebuf[p, pl.ds(gb, 16)] = e_vec
            for l in range(16):
                e_l = e_vec[l]
                for j in range(DOUT // 16):
                    embbuf[p, gb + l, pl.ds(j * 16, 16)] = (
                        embbuf[p, gb + l, pl.ds(j * 16, 16)] * e_l)

    def _chunk(c, p):
        _wait_in(c, p)
        _compute(c, p)
        _issue_scat(p)
        pprev = (p + 2) % NBUF
        @pl.when(c >= 1)
        def _():
            _wait_scat(pprev)
        _issue_in(c + 2, pprev)

    def _main(i, carry):
        for p in range(NBUF):
            _chunk(i * NBUF + p, p)
        return carry
    lax.fori_loop(0, MAIN // NBUF, _main, 0)

    for c in range(MAIN, CHUNKS):
        p = c % NBUF
        _wait_in(c, p)
        _compute(c, p)
        _wait_scat((p + 2) % NBUF)
        _issue_scat(p)
    _wait_scat((CHUNKS - 1) % NBUF)

    plsc.subcore_barrier()
    rows = pl.ds(sid * ROWS_PER_TILE, ROWS_PER_TILE)
    pltpu.sync_copy(q_shared.at[rows], q_hbm.at[cid, rows])
    pltpu.sync_copy(r_shared.at[rows], r_hbm.at[cid, rows])


def _post_body(q_ref, r_ref, h1_ref, a_ref, o_ref):
    a2 = a_ref[:, DIN:]
    qs = q_ref[0] + q_ref[1]
    r = r_ref[0] + r_ref[1]
    num = r * h1_ref[...] + lax.dot_general(
        qs, a2, (((1,), (1,)), ((), ())), preferred_element_type=jnp.float32)
    rt = jnp.where(r == 0.0, 1e-12, r)
    o_ref[...] = num / rt


def kernel(input, edge, edge_embed, a, a_2):
    src = edge[0, :]
    xp = jnp.pad(input, ((0, NPAD - N), (0, 0)))

    h1p, s_col = pl.pallas_call(
        _pre_body,
        out_shape=[
            jax.ShapeDtypeStruct((NPAD, DOUT), jnp.float32),
            jax.ShapeDtypeStruct((NPAD, 1), jnp.float32),
        ],
    )(xp, a, a_2)

    EB = 6400
    t_col = pl.pallas_call(
        _tmatvec_body,
        grid=(E // EB,),
        in_specs=[
            pl.BlockSpec((EB, DOUT), lambda i: (i, 0)),
            pl.BlockSpec((DIN, DIN + DOUT), lambda i: (0, 0)),
            pl.BlockSpec((1, DOUT), lambda i: (0, 0)),
        ],
        out_specs=pl.BlockSpec((1, EB), lambda i: (0, i)),
        out_shape=jax.ShapeDtypeStruct((1, E), jnp.float32),
    )(edge_embed, a, a_2)

    zq = jnp.zeros((ROWS_PER_TILE, DOUT), jnp.float32)
    zr = jnp.zeros((ROWS_PER_TILE,), jnp.float32)

    edge_pass = functools.partial(
        pl.kernel,
        out_type=[
            jax.ShapeDtypeStruct((NCORES, NPAD, DOUT), jnp.float32),
            jax.ShapeDtypeStruct((NCORES, NPAD), jnp.float32),
        ],
        mesh=plsc.VectorSubcoreMesh(
            core_axis_name="c", subcore_axis_name="s",
            num_cores=NCORES, num_subcores=NSUB),
        compiler_params=pltpu.CompilerParams(
            needs_layout_passes=False, use_tc_tiling_on_sc=False),
        scratch_types=[
            pltpu.VMEM((NBUF, B, DOUT), jnp.float32),
            pltpu.VMEM((NBUF, 1, B), jnp.int32),
            pltpu.VMEM((NBUF, B), jnp.float32),
            pltpu.VMEM((NBUF, B), jnp.float32),
            pltpu.VMEM((NPAD,), jnp.float32),
            pltpu.VMEM_SHARED((NPAD, DOUT), jnp.float32),
            pltpu.VMEM_SHARED((NPAD,), jnp.float32),
            pltpu.SemaphoreType.DMA,
            pltpu.SemaphoreType.DMA,
            pltpu.SemaphoreType.DMA,
            pltpu.SemaphoreType.DMA,
            pltpu.SemaphoreType.DMA,
            pltpu.SemaphoreType.DMA,
        ],
    )(_edge_body)
    q_all, r_all = edge_pass(edge_embed, src, s_col.reshape(NPAD),
                             t_col, zq, zr)

    out = pl.pallas_call(
        _post_body,
        out_shape=jax.ShapeDtypeStruct((NPAD, DOUT), jnp.float32),
    )(q_all, r_all.reshape(NCORES, NPAD, 1), h1p, a)
    return out[:N]
